# baseline (device time: 319166 ns/iter reference)
import jax
import jax.numpy as jnp
from jax import lax
from jax.experimental import pallas as pl
from jax.experimental.pallas import tpu as pltpu

N_Y = 4


def kernel(O, Wo):
    B, S, Hl, D = O.shape
    N = Wo.shape[1]
    S_out = S // N_Y

    K = Hl * D
    M = B * S_out

    def body(o_ref, w_ref, out_ref, comm_ref, acc_ref, send_sems, recv_sems):
        my_x = lax.axis_index("x")
        my_y = lax.axis_index("y")
        my_z = lax.axis_index("z")
        right = lax.rem(my_y + 1, N_Y)
        left = lax.rem(my_y + N_Y - 1, N_Y)

        barrier_sem = pltpu.get_barrier_semaphore()
        for nbr in (left, right):
            pl.semaphore_signal(
                barrier_sem, inc=1,
                device_id=(my_x, nbr, my_z),
                device_id_type=pl.DeviceIdType.MESH,
            )
        pl.semaphore_wait(barrier_sem, 2)

        def chunk_partial(c):
            o = o_ref[:, pl.ds(c * S_out, S_out), :, :]
            o2 = o.reshape(M, K)
            p = jnp.dot(o2, w_ref[...], preferred_element_type=jnp.float32)
            return p.reshape(B, S_out, N)

        acc_ref[...] = chunk_partial(lax.rem(my_y + N_Y - 1, N_Y))

        for s in range(N_Y - 1):
            rdma = pltpu.make_async_remote_copy(
                src_ref=acc_ref,
                dst_ref=comm_ref.at[s],
                send_sem=send_sems.at[s],
                recv_sem=recv_sems.at[s],
                device_id=(my_x, right, my_z),
                device_id_type=pl.DeviceIdType.MESH,
            )
            rdma.start()
            rdma.wait()
            c = lax.rem(my_y + 2 * N_Y - 2 - s, N_Y)
            if s < N_Y - 2:
                acc_ref[...] = chunk_partial(c) + comm_ref[s]
            else:
                out_ref[...] = chunk_partial(c) + comm_ref[s]

    return pl.pallas_call(
        body,
        out_shape=jax.ShapeDtypeStruct((B, S_out, N), jnp.float32),
        in_specs=[
            pl.BlockSpec(memory_space=pltpu.VMEM),
            pl.BlockSpec(memory_space=pltpu.VMEM),
        ],
        out_specs=pl.BlockSpec(memory_space=pltpu.VMEM),
        scratch_shapes=[
            pltpu.VMEM((N_Y - 1, B, S_out, N), jnp.float32),
            pltpu.VMEM((B, S_out, N), jnp.float32),
            pltpu.SemaphoreType.DMA((N_Y - 1,)),
            pltpu.SemaphoreType.DMA((N_Y - 1,)),
        ],
        compiler_params=pltpu.CompilerParams(
            collective_id=0,
            vmem_limit_bytes=100 * 1024 * 1024,
        ),
    )(O, Wo)


# device time: 301604 ns/iter; 1.0582x vs baseline; 1.0582x over previous
import jax
import jax.numpy as jnp
from jax import lax
from jax.experimental import pallas as pl
from jax.experimental.pallas import tpu as pltpu

N_Y = 4
Q = 2


def kernel(O, Wo):
    B, S, Hl, D = O.shape
    K = Hl * D
    N = Wo.shape[1]
    S_out = S // N_Y
    BQ = B // Q

    O2 = O.reshape(B, S, K)

    def body(o_ref, w_ref, out_ref, comm_ref, acc_ref, send_sems, recv_sems):
        my_x = lax.axis_index("x")
        my_y = lax.axis_index("y")
        my_z = lax.axis_index("z")
        right = lax.rem(my_y + 1, N_Y)
        left = lax.rem(my_y + N_Y - 1, N_Y)

        barrier_sem = pltpu.get_barrier_semaphore()
        for nbr in (left, right):
            pl.semaphore_signal(
                barrier_sem, inc=1,
                device_id=(my_x, nbr, my_z),
                device_id_type=pl.DeviceIdType.MESH,
            )
        pl.semaphore_wait(barrier_sem, 2)

        def sub_partial(c, q):
            o = o_ref[pl.ds(q * BQ, BQ), pl.ds(c * S_out, S_out), :]
            o2 = o.reshape(BQ * S_out, K)
            p = jnp.dot(o2, w_ref[...], preferred_element_type=jnp.float32)
            return p.reshape(BQ, S_out, N)

        def make_rdma(s, q):
            return pltpu.make_async_remote_copy(
                src_ref=acc_ref.at[pl.ds(q * BQ, BQ)],
                dst_ref=comm_ref.at[s, pl.ds(q * BQ, BQ)],
                send_sem=send_sems.at[s, q],
                recv_sem=recv_sems.at[s, q],
                device_id=(my_x, right, my_z),
                device_id_type=pl.DeviceIdType.MESH,
            )

        c0 = lax.rem(my_y + N_Y - 1, N_Y)
        for q in range(Q):
            acc_ref[pl.ds(q * BQ, BQ)] = sub_partial(c0, q)
            make_rdma(0, q).start()

        for s in range(N_Y - 1):
            c = lax.rem(my_y + 2 * N_Y - 2 - s, N_Y)
            for q in range(Q):
                make_rdma(s, q).wait()
                val = sub_partial(c, q) + comm_ref[s, pl.ds(q * BQ, BQ)]
                if s < N_Y - 2:
                    acc_ref[pl.ds(q * BQ, BQ)] = val
                    make_rdma(s + 1, q).start()
                else:
                    out_ref[pl.ds(q * BQ, BQ)] = val

    return pl.pallas_call(
        body,
        out_shape=jax.ShapeDtypeStruct((B, S_out, N), jnp.float32),
        in_specs=[
            pl.BlockSpec(memory_space=pltpu.VMEM),
            pl.BlockSpec(memory_space=pltpu.VMEM),
        ],
        out_specs=pl.BlockSpec(memory_space=pltpu.VMEM),
        scratch_shapes=[
            pltpu.VMEM((N_Y - 1, B, S_out, N), jnp.float32),
            pltpu.VMEM((B, S_out, N), jnp.float32),
            pltpu.SemaphoreType.DMA((N_Y - 1, Q)),
            pltpu.SemaphoreType.DMA((N_Y - 1, Q)),
        ],
        compiler_params=pltpu.CompilerParams(
            collective_id=0,
            vmem_limit_bytes=64 * 1024 * 1024,
        ),
    )(O2, Wo)
